# baseline (speedup 1.0000x reference)
import functools

import jax
import jax.numpy as jnp
from jax import lax
from jax.experimental import pallas as pl
from jax.experimental.pallas import tpu as pltpu
from jax.experimental.pallas import tpu_sc as plsc


def _logits_tile(x_ref, w1_ref, b1_ref, tb_ref, w2_ref, b2_ref, out_ref):
    h = jnp.dot(x_ref[...], w1_ref[...], preferred_element_type=jnp.float32)
    h = h + b1_ref[...]
    h = 0.5 * h * (1.0 + jax.lax.erf(h * 0.7071067811865476)) + tb_ref[...]
    logits = jnp.dot(h, w2_ref[...], preferred_element_type=jnp.float32)
    out_ref[...] = logits + b2_ref[...]


def _make_sc_route(n, c, rpw):
    mesh = plsc.VectorSubcoreMesh(core_axis_name="c", subcore_axis_name="s")

    @functools.partial(
        pl.kernel,
        mesh=mesh,
        out_type=jax.ShapeDtypeStruct((n, c), jnp.float32),
        scratch_types=[
            pltpu.VMEM((rpw, c), jnp.float32),
            pltpu.VMEM((rpw, c), jnp.float32),
        ],
    )
    def sc_route(logits_hbm, out_hbm, lg_v, pr_v):
        wid = lax.axis_index("s") * 2 + lax.axis_index("c")
        base = wid * rpw
        pltpu.sync_copy(logits_hbm.at[pl.ds(base, rpw)], lg_v)

        def argmax_bcast(vals, iota):
            v, ix = vals, iota
            sh = 1
            while sh < c:
                perm = iota ^ sh
                nv = v.at[perm].get(mode="promise_in_bounds")
                ni = ix.at[perm].get(mode="promise_in_bounds")
                better = (nv > v) | ((nv == v) & (ni < ix))
                v = jnp.where(better, nv, v)
                ix = jnp.where(better, ni, ix)
                sh *= 2
            return v, ix

        def body(i, carry):
            row = lg_v[i]
            iota = lax.iota(jnp.int32, c)
            m1, i1 = argmax_bcast(row, iota)
            hit1 = iota == i1
            m2, i2 = argmax_bcast(jnp.where(hit1, -jnp.inf, row), iota)
            hit2 = iota == i2
            p1 = 1.0 / (1.0 + jnp.exp(m2 - m1))
            p2 = 1.0 - p1
            pr_v[i] = jnp.where(hit1, p1, jnp.where(hit2, p2, 0.0))
            return carry

        lax.fori_loop(0, rpw, body, 0)
        pltpu.sync_copy(pr_v, out_hbm.at[pl.ds(base, rpw)])

    return sc_route


def kernel(x, W1, b1, W2, b2, task_table, task_id):
    original_shape = x.shape
    xf = x.reshape(-1, x.shape[-1])
    n, d = xf.shape
    e = W1.shape[1]
    c = W2.shape[1]
    tb = task_table[task_id].reshape(1, e)

    tm = 2048
    rep = lambda i: (0, 0)
    logits = pl.pallas_call(
        _logits_tile,
        grid=(n // tm,),
        in_specs=[
            pl.BlockSpec((tm, d), lambda i: (i, 0)),
            pl.BlockSpec((d, e), rep),
            pl.BlockSpec((1, e), rep),
            pl.BlockSpec((1, e), rep),
            pl.BlockSpec((e, c), rep),
            pl.BlockSpec((1, c), rep),
        ],
        out_specs=pl.BlockSpec((tm, c), lambda i: (i, 0)),
        out_shape=jax.ShapeDtypeStruct((n, c), jnp.float32),
    )(xf, W1, b1.reshape(1, e), tb, W2, b2.reshape(1, c))

    probs = _make_sc_route(n, c, n // 32)(logits)
    return probs.reshape(*original_shape[:-1], c)


# speedup vs baseline: 1.6420x; 1.6420x over previous
import jax
import jax.numpy as jnp
from jax.experimental import pallas as pl


def _router_tile(x_ref, w1_ref, b1_ref, tb_ref, w2_ref, b2_ref, out_ref):
    h = jnp.dot(x_ref[...], w1_ref[...], preferred_element_type=jnp.float32)
    h = h + b1_ref[...]
    h = 0.5 * h * (1.0 + jax.lax.erf(h * 0.7071067811865476)) + tb_ref[...]
    logits = jnp.dot(h, w2_ref[...], preferred_element_type=jnp.float32)
    logits = logits + b2_ref[...]

    c = logits.shape[-1]
    iota = jax.lax.broadcasted_iota(jnp.int32, logits.shape, 1)
    m1 = jnp.max(logits, axis=-1, keepdims=True)
    idx1 = jnp.min(jnp.where(logits == m1, iota, c), axis=-1, keepdims=True)
    hit1 = iota == idx1
    masked = jnp.where(hit1, -jnp.inf, logits)
    m2 = jnp.max(masked, axis=-1, keepdims=True)
    idx2 = jnp.min(jnp.where(masked == m2, iota, c), axis=-1, keepdims=True)
    hit2 = iota == idx2
    e2 = jnp.exp(m2 - m1)
    p1 = 1.0 / (1.0 + e2)
    p2 = 1.0 - p1
    out_ref[...] = jnp.where(hit1, p1, jnp.where(hit2, p2, 0.0))


def kernel(x, W1, b1, W2, b2, task_table, task_id):
    original_shape = x.shape
    xf = x.reshape(-1, x.shape[-1])
    n, d = xf.shape
    e = W1.shape[1]
    c = W2.shape[1]
    tb = task_table[task_id].reshape(1, e)

    tm = 2048
    rep = lambda i: (0, 0)
    probs = pl.pallas_call(
        _router_tile,
        grid=(n // tm,),
        in_specs=[
            pl.BlockSpec((tm, d), lambda i: (i, 0)),
            pl.BlockSpec((d, e), rep),
            pl.BlockSpec((1, e), rep),
            pl.BlockSpec((1, e), rep),
            pl.BlockSpec((e, c), rep),
            pl.BlockSpec((1, c), rep),
        ],
        out_specs=pl.BlockSpec((tm, c), lambda i: (i, 0)),
        out_shape=jax.ShapeDtypeStruct((n, c), jnp.float32),
    )(xf, W1, b1.reshape(1, e), tb, W2, b2.reshape(1, c))
    return probs.reshape(*original_shape[:-1], c)
